# baseline (speedup 1.0000x reference)
import functools

import jax
import jax.numpy as jnp
from jax import lax
from jax.experimental import pallas as pl
from jax.experimental.pallas import tpu as pltpu
from jax.experimental.pallas import tpu_sc as plsc

RPG = 4
SLEN = 100
NBUF = 3


@functools.lru_cache(maxsize=None)
def _build(batch, hist, vocab, dim):
    mesh = plsc.VectorSubcoreMesh(core_axis_name="c", subcore_axis_name="s")
    nw = mesh.num_cores * mesh.num_subcores
    assert batch % (nw * RPG) == 0
    chunks, c = [], 0
    while c < hist:
        ln = min(128, hist - c)
        chunks.append((c, ln))
        c += ln
    rows_per_worker = batch // nw
    groups = rows_per_worker // RPG
    assert groups >= 7

    @functools.partial(
        pl.kernel,
        mesh=mesh,
        out_type=jax.ShapeDtypeStruct((batch, hist, dim), jnp.float32),
        scratch_types=[
            pltpu.VMEM((NBUF, RPG, hist), jnp.int32),
            pltpu.VMEM((NBUF, RPG, hist, dim), jnp.float32),
            [pltpu.SemaphoreType.DMA] * NBUF,
            [pltpu.SemaphoreType.DMA] * NBUF,
            [pltpu.SemaphoreType.DMA] * NBUF,
        ],
        compiler_params=pltpu.CompilerParams(use_tc_tiling_on_sc=False),
    )
    def k(idx_hbm, table_hbm, out_hbm, idx_v, rows_v, isem, gsem, osem):
        wid = lax.axis_index("s") * mesh.num_cores + lax.axis_index("c")
        row0 = wid * rows_per_worker

        def fire_idx(g, b):
            pltpu.async_copy(
                idx_hbm.at[pl.ds(row0 + g * RPG, RPG)], idx_v.at[b], isem[b]
            )

        def drain_idx(b):
            pltpu.make_async_copy(
                idx_hbm.at[pl.ds(0, RPG)], idx_v.at[b], isem[b]
            ).wait()

        def fire_gathers(g, b):
            for r in range(RPG):
                for c, ln in chunks:
                    pltpu.async_copy(
                        table_hbm.at[idx_v.at[b, r, pl.ds(c, ln)]],
                        rows_v.at[b, r, pl.ds(c, ln)],
                        gsem[b],
                    )

        def drain_gathers(b):
            for r in range(RPG):
                for c, ln in chunks:
                    pltpu.make_async_copy(
                        out_hbm.at[0, pl.ds(0, ln)],
                        rows_v.at[b, r, pl.ds(c, ln)],
                        gsem[b],
                    ).wait()

        def fire_out(g, b):
            pltpu.async_copy(
                rows_v.at[b], out_hbm.at[pl.ds(row0 + g * RPG, RPG)], osem[b]
            )

        def drain_out(b):
            pltpu.make_async_copy(
                rows_v.at[b], out_hbm.at[pl.ds(0, RPG)], osem[b]
            ).wait()

        def steady(u, b, guard_idx):
            bm2 = (b + 1) % NBUF
            drain_gathers(bm2)
            fire_out(u - 2, bm2)
            if guard_idx:
                @pl.when(u + 1 < groups)
                def _():
                    fire_idx(u + 1, bm2)
            else:
                fire_idx(u + 1, bm2)
            drain_out(b)
            drain_idx(b)
            fire_gathers(u, b)

        fire_idx(0, 0)
        fire_idx(1, 1)
        fire_idx(2, 2)
        drain_idx(0)
        fire_gathers(0, 0)
        drain_idx(1)
        fire_gathers(1, 1)
        drain_gathers(0)
        fire_out(0, 0)
        fire_idx(3, 0)
        drain_idx(2)
        fire_gathers(2, 2)
        steady(3, 0, guard_idx=False)

        t0 = 4 + (groups - 4) % NBUF
        for u in range(4, t0):
            steady(u, u % NBUF, guard_idx=False)

        def step(s, carry):
            t = NBUF * s + t0
            for o in range(NBUF):
                steady(t + o, (t0 + o) % NBUF, True)
            return carry

        lax.fori_loop(0, (groups - t0) // NBUF, step, 0)

        drain_gathers((groups - 2) % NBUF)
        fire_out(groups - 2, (groups - 2) % NBUF)
        drain_gathers((groups - 1) % NBUF)
        fire_out(groups - 1, (groups - 1) % NBUF)
        drain_out((groups - 3) % NBUF)
        drain_out((groups - 2) % NBUF)
        drain_out((groups - 1) % NBUF)

    return k


def kernel(inputs, table):
    b, h = inputs.shape
    vocab, dim = table.shape
    return _build(b, h, vocab, dim)(inputs.astype(jnp.int32), table)
